# Initial kernel scaffold; baseline (speedup 1.0000x reference)
import jax
import jax.numpy as jnp
from jax.experimental import pallas as pl


def kernel(features, cic_scores):
    raise NotImplementedError("write your pallas kernel here")


# speedup vs baseline: 1.8907x; 1.8907x over previous
import functools

import jax
import jax.numpy as jnp
from jax import lax
from jax.experimental import pallas as pl
from jax.experimental.pallas import tpu as pltpu
from jax.experimental.pallas import tpu_sc as plsc

N = 100000
D = 256
K = 20000
NW = 32
PW = 640
P = NW * PW
CH = 128
NCH = PW // CH


def _perturb_sc(out_ref, features, idxp, donorp, noisehp):
    mesh = plsc.VectorSubcoreMesh(core_axis_name="c", subcore_axis_name="s")

    @functools.partial(
        pl.kernel,
        mesh=mesh,
        scratch_types=[
            pltpu.VMEM((CH,), jnp.int32),
            pltpu.VMEM((CH,), jnp.int32),
            pltpu.VMEM((CH, D), jnp.float32),
            pltpu.VMEM((CH, D), jnp.float32),
            pltpu.VMEM((CH, D), jnp.float32),
            pltpu.SemaphoreType.DMA,
            pltpu.SemaphoreType.DMA,
        ],
    )
    def k(out_hbm, feat_hbm, idx_hbm, donor_hbm, noise_hbm,
          idx_v, don_v, g_v, d_v, n_v, sem1, sem2):
        wid = lax.axis_index("s") * 2 + lax.axis_index("c")
        base0 = wid * PW

        def chunk(c, carry):
            base = base0 + c * CH
            pltpu.sync_copy(idx_hbm.at[pl.ds(base, CH)], idx_v)
            pltpu.sync_copy(donor_hbm.at[pl.ds(base, CH)], don_v)
            cp1 = pltpu.async_copy(feat_hbm.at[idx_v], g_v, sem1)
            cp2 = pltpu.async_copy(feat_hbm.at[don_v], d_v, sem2)
            pltpu.sync_copy(noise_hbm.at[pl.ds(base, CH), :], n_v)
            cp1.wait()
            cp2.wait()

            def row(r, carry2):
                for cc in range(D // 16):
                    sl = (r, pl.ds(cc * 16, 16))
                    g_v[sl] = (g_v[sl] + d_v[sl]) * 0.5 + n_v[sl]
                return carry2

            lax.fori_loop(0, CH, row, 0)
            pltpu.async_copy(g_v, out_hbm.at[idx_v], sem1).wait()
            return carry

        lax.fori_loop(0, NCH, chunk, 0)

    k(out_ref, features, idxp, donorp, noisehp)


def kernel(features, cic_scores):
    n, d = features.shape

    scores = cic_scores.sum(axis=1)
    _, idx = lax.top_k(-scores, K)

    rkey = jax.random.key(42)
    k_noise, k_donor = jax.random.split(rkey)
    noiseh = jax.random.normal(k_noise, (K, d), jnp.float32) * 0.25
    donor = jax.random.randint(k_donor, (K,), 0, n)

    idxp = jnp.concatenate([idx, jnp.broadcast_to(idx[K - 1], (P - K,))])
    donorp = jnp.concatenate([donor, jnp.broadcast_to(donor[K - 1], (P - K,))])
    noisehp = jnp.concatenate(
        [noiseh, jnp.broadcast_to(noiseh[K - 1], (P - K, d))])

    out_ref = jax.new_ref(features)
    _perturb_sc(out_ref, features, idxp, donorp, noisehp)
    return out_ref[...]
